# baseline (speedup 1.0000x reference)
import functools

import jax
import jax.numpy as jnp
from jax import lax
from jax.experimental import pallas as pl
from jax.experimental.pallas import tpu as pltpu
from jax.experimental.pallas import tpu_sc as plsc

VOCAB_TILE = 1024


def _gather_sc(emb, idx):
    B = idx.shape[0]
    V, D = emb.shape
    info = plsc.get_sparse_core_info()
    nc, ns = info.num_cores, info.num_subcores
    nw = nc * ns
    b_per_w = B // nw
    mesh = plsc.VectorSubcoreMesh(core_axis_name="c", subcore_axis_name="s")

    @functools.partial(
        pl.kernel,
        mesh=mesh,
        out_type=jax.ShapeDtypeStruct((B, D), jnp.float32),
        scratch_types=[
            pltpu.VMEM((b_per_w,), jnp.int32),
            pltpu.VMEM((b_per_w, D), jnp.float32),
            pltpu.SemaphoreType.DMA,
        ],
        compiler_params=pltpu.CompilerParams(use_tc_tiling_on_sc=False),
    )
    def gather(table_hbm, idx_hbm, out_hbm, idx_v, rows_v, sem):
        wid = lax.axis_index("s") * nc + lax.axis_index("c")
        base = wid * b_per_w
        pltpu.sync_copy(idx_hbm.at[pl.ds(base, b_per_w)], idx_v)
        pltpu.async_copy(table_hbm.at[idx_v], rows_v, sem).wait()
        pltpu.sync_copy(rows_v, out_hbm.at[pl.ds(base, b_per_w)])

    return gather(emb, idx)


def _mlp_tc(x, W1, b1, W2, b2):
    B, D = x.shape
    V = W2.shape[1]
    nt = pl.cdiv(V, VOCAB_TILE)

    def body(x_ref, w1_ref, b1_ref, w2_ref, b2_ref, out_ref, h_ref):
        @pl.when(pl.program_id(0) == 0)
        def _():
            xw = jnp.dot(x_ref[...], w1_ref[...],
                         preferred_element_type=jnp.float32)
            h_ref[...] = jnp.maximum(xw + b1_ref[...], 0.0)

        hw = jnp.dot(h_ref[...], w2_ref[...],
                     preferred_element_type=jnp.float32)
        out_ref[...] = hw + b2_ref[...]

    return pl.pallas_call(
        body,
        grid=(nt,),
        in_specs=[
            pl.BlockSpec((B, D), lambda j: (0, 0)),
            pl.BlockSpec((D, D), lambda j: (0, 0)),
            pl.BlockSpec((1, D), lambda j: (0, 0)),
            pl.BlockSpec((D, VOCAB_TILE), lambda j: (0, j)),
            pl.BlockSpec((1, VOCAB_TILE), lambda j: (0, j)),
        ],
        out_specs=pl.BlockSpec((B, VOCAB_TILE), lambda j: (0, j)),
        out_shape=jax.ShapeDtypeStruct((B, V), jnp.float32),
        scratch_shapes=[pltpu.VMEM((B, D), jnp.float32)],
    )(x, W1, b1.reshape(1, D), W2, b2.reshape(1, V))


def kernel(idx, emb, W1, b1, W2, b2):
    x = _gather_sc(emb, idx)
    return _mlp_tc(x, W1, b1, W2, b2)


# speedup vs baseline: 1.0326x; 1.0326x over previous
import functools

import jax
import jax.numpy as jnp
from jax import lax
from jax.experimental import pallas as pl
from jax.experimental.pallas import tpu as pltpu
from jax.experimental.pallas import tpu_sc as plsc

VOCAB_TILE = 1024


def _gather_sc(emb, idx):
    B = idx.shape[0]
    V, D = emb.shape
    info = plsc.get_sparse_core_info()
    nc, ns = info.num_cores, info.num_subcores
    nw = nc * ns
    b_per_w = B // nw
    mesh = plsc.VectorSubcoreMesh(core_axis_name="c", subcore_axis_name="s")

    @functools.partial(
        pl.kernel,
        mesh=mesh,
        out_type=jax.ShapeDtypeStruct((B, D), jnp.float32),
        scratch_types=[
            pltpu.VMEM((b_per_w,), jnp.int32),
            pltpu.VMEM((b_per_w, D), jnp.float32),
            pltpu.SemaphoreType.DMA,
        ],
        compiler_params=pltpu.CompilerParams(use_tc_tiling_on_sc=False),
    )
    def gather(table_hbm, idx_hbm, out_hbm, idx_v, rows_v, sem):
        wid = lax.axis_index("s") * nc + lax.axis_index("c")
        base = wid * b_per_w
        pltpu.sync_copy(idx_hbm.at[pl.ds(base, b_per_w)], idx_v)
        pltpu.async_copy(table_hbm.at[idx_v], rows_v, sem).wait()
        pltpu.sync_copy(rows_v, out_hbm.at[pl.ds(base, b_per_w)])

    return gather(emb, idx)


def _mlp_tc(x, W1, b1, W2, b2):
    B, D = x.shape
    V = W2.shape[1]
    nt = pl.cdiv(V, VOCAB_TILE)

    def body(x_ref, w1_ref, b1_ref, w2_ref, b2_ref, out_ref, h_ref):
        @pl.when(pl.program_id(0) == 0)
        def _():
            xw = jnp.dot(x_ref[...], w1_ref[...],
                         preferred_element_type=jnp.float32)
            h_ref[...] = jnp.maximum(xw + b1_ref[...], 0.0)

        hw = jnp.dot(h_ref[...], w2_ref[...],
                     preferred_element_type=jnp.float32)
        out_ref[...] = hw + b2_ref[...]

    return pl.pallas_call(
        body,
        grid=(nt,),
        in_specs=[
            pl.BlockSpec((B, D), lambda j: (0, 0)),
            pl.BlockSpec((D, D), lambda j: (0, 0)),
            pl.BlockSpec((1, D), lambda j: (0, 0)),
            pl.BlockSpec((D, VOCAB_TILE), lambda j: (0, j)),
            pl.BlockSpec((1, VOCAB_TILE), lambda j: (0, j)),
        ],
        out_specs=pl.BlockSpec((B, VOCAB_TILE), lambda j: (0, j)),
        out_shape=jax.ShapeDtypeStruct((B, V), jnp.float32),
        scratch_shapes=[pltpu.VMEM((B, D), jnp.float32)],
    )(x, W1, b1.reshape(1, D), W2, b2.reshape(1, V))


def kernel(idx, emb, W1, b1, W2, b2):
    x = jnp.take(emb, idx, axis=0)
    return _mlp_tc(x, W1, b1, W2, b2)
